# Initial kernel scaffold; baseline (speedup 1.0000x reference)
import jax
import jax.numpy as jnp
from jax.experimental import pallas as pl


def kernel(nodes_list, id2emb):
    raise NotImplementedError("write your pallas kernel here")


# speedup vs baseline: 1.1307x; 1.1307x over previous
import functools

import jax
import jax.numpy as jnp
from jax import lax
from jax.experimental import pallas as pl
from jax.experimental.pallas import tpu as pltpu
from jax.experimental.pallas import tpu_sc as plsc

D = 128
NC = 2
NS = 16
NW = NC * NS
CH = 128


@functools.partial(jax.jit, static_argnums=(0,))
def _gather(nch, idx, table):
    B = NW * nch * CH
    mesh = plsc.VectorSubcoreMesh(core_axis_name="c", subcore_axis_name="s")

    @functools.partial(
        pl.kernel,
        mesh=mesh,
        out_type=jax.ShapeDtypeStruct((B, D), jnp.float32),
        scratch_types=[
            pltpu.VMEM((nch, CH), jnp.int32),
            pltpu.VMEM((CH, D), jnp.float32),
            pltpu.SemaphoreType.DMA,
        ],
    )
    def k(table_hbm, idx_hbm, out_hbm, idx_v, rows_v, sem):
        wid = lax.axis_index("s") * NC + lax.axis_index("c")
        pltpu.sync_copy(idx_hbm.at[wid], idx_v)
        base = wid * (nch * CH)

        def body(j, carry):
            pltpu.async_copy(table_hbm.at[idx_v.at[j]], rows_v, sem).wait()
            pltpu.sync_copy(rows_v, out_hbm.at[pl.ds(base + j * CH, CH)])
            return carry

        lax.fori_loop(0, nch, body, 0)

    return k(table, idx)


def kernel(nodes_list, id2emb):
    batch, hist = nodes_list.shape
    B = batch * hist
    assert B % (NW * CH) == 0
    nch = B // (NW * CH)
    idx = nodes_list.astype(jnp.int32).reshape(NW, nch, CH)
    out = _gather(nch, idx, id2emb)
    return out.reshape(batch, hist, D)


# speedup vs baseline: 1.2792x; 1.1314x over previous
import functools

import jax
import jax.numpy as jnp
from jax import lax
from jax.experimental import pallas as pl
from jax.experimental.pallas import tpu as pltpu
from jax.experimental.pallas import tpu_sc as plsc

D = 128
NC = 2
NS = 16
NW = NC * NS
CH = 128
NBUF = 4


@functools.partial(jax.jit, static_argnums=(0,))
def _gather(nch, idx, table):
    B = NW * nch * CH
    assert nch % NBUF == 0
    nouter = nch // NBUF
    mesh = plsc.VectorSubcoreMesh(core_axis_name="c", subcore_axis_name="s")

    @functools.partial(
        pl.kernel,
        mesh=mesh,
        out_type=jax.ShapeDtypeStruct((B, D), jnp.float32),
        scratch_types=[
            pltpu.VMEM((nch, CH), jnp.int32),
            pltpu.VMEM((NBUF, CH, D), jnp.float32),
            pltpu.SemaphoreType.DMA((NBUF,)),
            pltpu.SemaphoreType.DMA((NBUF,)),
        ],
    )
    def k(table_hbm, idx_hbm, out_hbm, idx_v, rows_v, gsem, osem):
        wid = lax.axis_index("s") * NC + lax.axis_index("c")
        pltpu.sync_copy(idx_hbm.at[wid], idx_v)
        base = wid * (nch * CH)

        for b in range(NBUF):
            pltpu.async_copy(table_hbm.at[idx_v.at[b]], rows_v.at[b], gsem.at[b])

        def outer(i, carry):
            g0 = i * NBUF
            for b in range(NBUF):
                g = g0 + b
                pltpu.make_async_copy(
                    table_hbm.at[idx_v.at[g]], rows_v.at[b], gsem.at[b]
                ).wait()
                dst = out_hbm.at[pl.ds(base + g * CH, CH)]
                pltpu.async_copy(rows_v.at[b], dst, osem.at[b])
                pltpu.make_async_copy(rows_v.at[b], dst, osem.at[b]).wait()
                pltpu.async_copy(
                    table_hbm.at[idx_v.at[g + NBUF]], rows_v.at[b], gsem.at[b]
                )
            return carry

        lax.fori_loop(0, nouter - 1, outer, 0)

        for b in range(NBUF):
            g = nch - NBUF + b
            pltpu.make_async_copy(
                table_hbm.at[idx_v.at[g]], rows_v.at[b], gsem.at[b]
            ).wait()
            dst = out_hbm.at[pl.ds(base + g * CH, CH)]
            pltpu.async_copy(rows_v.at[b], dst, osem.at[b])
            pltpu.make_async_copy(rows_v.at[b], dst, osem.at[b]).wait()

    return k(table, idx)


def kernel(nodes_list, id2emb):
    batch, hist = nodes_list.shape
    B = batch * hist
    assert B % (NW * CH) == 0
    nch = B // (NW * CH)
    idx = nodes_list.astype(jnp.int32).reshape(NW, nch, CH)
    out = _gather(nch, idx, id2emb)
    return out.reshape(batch, hist, D)
